# baseline (speedup 1.0000x reference)
import functools

import jax
import jax.numpy as jnp
from jax import lax
from jax.experimental import pallas as pl
from jax.experimental.pallas import tpu as pltpu
from jax.experimental.pallas import tpu_sc as plsc

_G = 128


@functools.cache
def _spmm_partials_kernel(n, d, e_pad):
    info = plsc.get_sparse_core_info()
    nc, ns = info.num_cores, info.num_subcores
    nw = nc * ns
    n_chunks = e_pad // _G
    nch = n_chunks // nw
    assert nch * nw * _G == e_pad and nch % 2 == 0
    rpt = (n // ns) & ~7
    last_rows = n - rpt * (ns - 1)
    assert rpt % 8 == 0 and last_rows > 0
    mesh = plsc.VectorSubcoreMesh(core_axis_name="c", subcore_axis_name="s")

    @functools.partial(
        pl.kernel,
        out_type=jax.ShapeDtypeStruct((nc, n, d), jnp.float32),
        mesh=mesh,
        scratch_types=[
            pltpu.VMEM_SHARED((n, d), jnp.float32),
            pltpu.VMEM((nch, _G), jnp.int32),
            pltpu.VMEM((_G,), jnp.int32),
            pltpu.VMEM((_G,), jnp.int32),
            pltpu.VMEM((_G,), jnp.float32),
            pltpu.VMEM((_G,), jnp.float32),
            pltpu.VMEM((_G, d), jnp.float32),
            pltpu.VMEM((_G, d), jnp.float32),
            pltpu.SemaphoreType.DMA,
            pltpu.SemaphoreType.DMA,
            pltpu.SemaphoreType.DMA,
            pltpu.SemaphoreType.DMA,
            pltpu.SemaphoreType.DMA,
            pltpu.SemaphoreType.DMA,
        ],
    )
    def spmm(table_hbm, src_hbm, dst_hbm, w_hbm, zeros_hbm, out_hbm,
             acc, srcb, dstb0, dstb1, wb0, wb1, rows0, rows1,
             gsem0, gsem1, dsem0, dsem1, wsem0, wsem1):
        c = lax.axis_index("c")
        s = lax.axis_index("s")
        wid = c * ns + s
        rows = (rows0, rows1)
        dstb = (dstb0, dstb1)
        wb = (wb0, wb1)
        gsem = (gsem0, gsem1)
        dsem = (dsem0, dsem1)
        wsem = (wsem0, wsem1)

        c0 = wid * nch
        pltpu.sync_copy(src_hbm.at[pl.ds(c0, nch)], srcb)

        r0 = s * rpt

        @pl.when(s < ns - 1)
        def _():
            pltpu.sync_copy(zeros_hbm.at[pl.ds(r0, rpt)],
                            acc.at[pl.ds(r0, rpt)])

        @pl.when(s == ns - 1)
        def _():
            pltpu.sync_copy(zeros_hbm.at[pl.ds(r0, last_rows)],
                            acc.at[pl.ds(r0, last_rows)])

        plsc.subcore_barrier()

        pltpu.async_copy(dst_hbm.at[c0], dstb0, dsem0)
        pltpu.async_copy(w_hbm.at[c0], wb0, wsem0)
        pltpu.async_copy(dst_hbm.at[c0 + 1], dstb1, dsem1)
        pltpu.async_copy(w_hbm.at[c0 + 1], wb1, wsem1)
        pltpu.async_copy(table_hbm.at[srcb.at[0]], rows0, gsem0)

        def pair_body(i, carry):
            for u in range(2):
                j = 2 * i + u
                b1 = 1 - u
                rb, db, wbuf = rows[u], dstb[u], wb[u]
                pltpu.make_async_copy(table_hbm.at[srcb.at[j]], rb,
                                      gsem[u]).wait()

                @pl.when((u == 0) | (i < nch // 2 - 1))
                def _():
                    pltpu.async_copy(table_hbm.at[srcb.at[j + 1]],
                                     rows[b1], gsem[b1])

                pltpu.make_async_copy(dst_hbm.at[c0 + j], db, dsem[u]).wait()
                pltpu.make_async_copy(w_hbm.at[c0 + j], wbuf, wsem[u]).wait()

                def group_body(g, carry2):
                    wvec = wbuf[pl.ds(g * 16, 16)]
                    for l in range(16):
                        wv = wvec[l]
                        t = g * 16 + l
                        for k in range(d // 16):
                            sl = pl.ds(k * 16, 16)
                            rb[t, sl] = rb[t, sl] * wv
                    return carry2

                lax.fori_loop(0, _G // 16, group_body, 0)
                pltpu.sync_copy(rb, acc.at[db], add=True)

                @pl.when(i < nch // 2 - 1)
                def _():
                    pltpu.async_copy(dst_hbm.at[c0 + j + 2], db, dsem[u])
                    pltpu.async_copy(w_hbm.at[c0 + j + 2], wbuf, wsem[u])

            return carry

        lax.fori_loop(0, nch // 2, pair_body, 0)
        plsc.subcore_barrier()

        @pl.when(s < ns - 1)
        def _():
            pltpu.sync_copy(acc.at[pl.ds(r0, rpt)],
                            out_hbm.at[c, pl.ds(r0, rpt)])

        @pl.when(s == ns - 1)
        def _():
            pltpu.sync_copy(acc.at[pl.ds(r0, last_rows)],
                            out_hbm.at[c, pl.ds(r0, last_rows)])

    return spmm


def _tc_stage1(p, wcat, bcat):
    _, n, d = p.shape
    blk = 400
    assert n % blk == 0

    def body(p_ref, w_ref, b_ref, h1_ref, h2_ref):
        h = p_ref[0] + p_ref[1]
        y = jnp.dot(h, w_ref[...], preferred_element_type=jnp.float32)
        y = jnp.maximum(y + b_ref[...], 0.0)
        h1_ref[...] = y[:, :d]
        h2_ref[...] = y[:, d:]

    return pl.pallas_call(
        body,
        grid=(n // blk,),
        in_specs=[
            pl.BlockSpec((2, blk, d), lambda i: (0, i, 0)),
            pl.BlockSpec((d, 2 * d), lambda i: (0, 0)),
            pl.BlockSpec((2 * d,), lambda i: (0,)),
        ],
        out_specs=[
            pl.BlockSpec((blk, d), lambda i: (i, 0)),
            pl.BlockSpec((blk, d), lambda i: (i, 0)),
        ],
        out_shape=[
            jax.ShapeDtypeStruct((n, d), jnp.float32),
            jax.ShapeDtypeStruct((n, d), jnp.float32),
        ],
    )(p, wcat, bcat)


def _tc_stage2(pb, pc, w1, b1, w2, b2):
    _, n, d = pb.shape
    blk = 400
    assert n % blk == 0

    def body(pb_ref, pc_ref, w1_ref, b1_ref, w2_ref, b2_ref, x1_ref, x2_ref):
        s1 = pb_ref[0] + pb_ref[1]
        s2 = pc_ref[0] + pc_ref[1]
        x1_ref[...] = jnp.dot(s1, w1_ref[...],
                              preferred_element_type=jnp.float32) + b1_ref[...]
        x2_ref[...] = jnp.dot(s2, w2_ref[...],
                              preferred_element_type=jnp.float32) + b2_ref[...]

    return pl.pallas_call(
        body,
        grid=(n // blk,),
        in_specs=[
            pl.BlockSpec((2, blk, d), lambda i: (0, i, 0)),
            pl.BlockSpec((2, blk, d), lambda i: (0, i, 0)),
            pl.BlockSpec((d, d), lambda i: (0, 0)),
            pl.BlockSpec((d,), lambda i: (0,)),
            pl.BlockSpec((d, d), lambda i: (0, 0)),
            pl.BlockSpec((d,), lambda i: (0,)),
        ],
        out_specs=[
            pl.BlockSpec((blk, d), lambda i: (i, 0)),
            pl.BlockSpec((blk, d), lambda i: (i, 0)),
        ],
        out_shape=[
            jax.ShapeDtypeStruct((n, d), jnp.float32),
            jax.ShapeDtypeStruct((n, d), jnp.float32),
        ],
    )(pb, pc, w1, b1, w2, b2)


def kernel(x, edge_index, edge_weight, W1_0, b1_0, W1_1, b1_1,
           W2_0, b2_0, W2_1, b2_1):
    n, d = x.shape
    e = edge_weight.shape[0]

    info = plsc.get_sparse_core_info()
    nw = info.num_cores * info.num_subcores
    quantum = nw * _G * 2
    e_pad = ((e + quantum - 1) // quantum) * quantum
    pad = e_pad - e
    dst = jnp.pad(edge_index[0], (0, pad)).reshape(e_pad // _G, _G)
    src = jnp.pad(edge_index[1], (0, pad)).reshape(e_pad // _G, _G)
    w = jnp.pad(edge_weight, (0, pad)).reshape(e_pad // _G, _G)
    zeros = jnp.zeros((n, d), jnp.float32)

    spmm = _spmm_partials_kernel(n, d, e_pad)
    pa = spmm(x, src, dst, w, zeros)

    wcat = jnp.concatenate([W1_0, W2_0], axis=1)
    bcat = jnp.concatenate([b1_0, b2_0], axis=0)
    h1, h2 = _tc_stage1(pa, wcat, bcat)

    pb = spmm(h1, src, dst, w, zeros)
    pc = spmm(h2, src, dst, w, zeros)

    x1, x2 = _tc_stage2(pb, pc, W1_1, b1_1, W2_1, b2_1)
    return (x1, x2)


# speedup vs baseline: 2.5652x; 2.5652x over previous
import functools

import jax
import jax.numpy as jnp
from jax import lax
from jax.experimental import pallas as pl
from jax.experimental.pallas import tpu as pltpu
from jax.experimental.pallas import tpu_sc as plsc

_G = 128


@functools.cache
def _spmm_partials_kernel(n, d, e_pad):
    info = plsc.get_sparse_core_info()
    nc, ns = info.num_cores, info.num_subcores
    nw = nc * ns
    n_chunks = e_pad // _G
    nch = n_chunks // nw
    assert nch * nw * _G == e_pad and nch % 4 == 0
    rpt = (n // ns) & ~7
    last_rows = n - rpt * (ns - 1)
    assert rpt % 8 == 0 and last_rows > 0
    mesh = plsc.VectorSubcoreMesh(core_axis_name="c", subcore_axis_name="s")

    half = _G // 2
    scratch = (
        [pltpu.VMEM_SHARED((n, d), jnp.float32)]
        + [pltpu.VMEM((nch, _G), jnp.int32)]
        + [pltpu.VMEM((2, half), jnp.int32)] * 4
        + [pltpu.VMEM((_G,), jnp.float32)] * 2
        + [pltpu.VMEM((_G, d), jnp.float32)] * 2
        + [pltpu.SemaphoreType.DMA] * 10
    )

    @functools.partial(
        pl.kernel,
        out_type=jax.ShapeDtypeStruct((nc, n, d), jnp.float32),
        mesh=mesh,
        scratch_types=scratch,
    )
    def spmm(table_hbm, src_hbm, dst_hbm, w_hbm, zeros_hbm, out_hbm,
             acc, srcb, dstb0, dstb1, dstb2, dstb3, wb0, wb1, rows0, rows1,
             gsem0, gsem1, ssem0, ssem1, dsem0, dsem1, dsem2, dsem3,
             wsem0, wsem1):
        c = lax.axis_index("c")
        s = lax.axis_index("s")
        wid = c * ns + s
        rows = (rows0, rows1)
        dstb = (dstb0, dstb1, dstb2, dstb3)
        wb = (wb0, wb1)
        gsem = (gsem0, gsem1)
        ssem = (ssem0, ssem1)
        dsem = (dsem0, dsem1, dsem2, dsem3)
        wsem = (wsem0, wsem1)

        c0 = wid * nch
        pltpu.sync_copy(src_hbm.at[pl.ds(c0, nch)], srcb)

        r0 = s * rpt

        @pl.when(s < ns - 1)
        def _():
            pltpu.sync_copy(zeros_hbm.at[pl.ds(r0, rpt)],
                            acc.at[pl.ds(r0, rpt)])

        @pl.when(s == ns - 1)
        def _():
            pltpu.sync_copy(zeros_hbm.at[pl.ds(r0, last_rows)],
                            acc.at[pl.ds(r0, last_rows)])

        plsc.subcore_barrier()

        def scale_half(rb, wbuf, h):
            def group_body(g, carry2):
                wvec = wbuf[pl.ds(h * half + g * 16, 16)]
                for l in range(16):
                    wv = wvec[l]
                    t = h * half + g * 16 + l
                    for k in range(d // 16):
                        sl = pl.ds(k * 16, 16)
                        rb[t, sl] = rb[t, sl] * wv
                return carry2

            lax.fori_loop(0, half // 16, group_body, 0)

        def scat_start(rb, db, h, sem):
            pltpu.async_copy(rb.at[pl.ds(h * half, half)], acc.at[db.at[h]],
                             sem, add=True)

        def scat_wait(rb, db, h, sem):
            pltpu.make_async_copy(rb.at[pl.ds(h * half, half)],
                                  acc.at[db.at[h]], sem).wait()

        pltpu.async_copy(dst_hbm.at[c0], dstb0, dsem0)
        pltpu.async_copy(dst_hbm.at[c0 + 1], dstb1, dsem1)
        pltpu.async_copy(dst_hbm.at[c0 + 2], dstb2, dsem2)
        pltpu.async_copy(w_hbm.at[c0], wb0, wsem0)
        pltpu.async_copy(w_hbm.at[c0 + 1], wb1, wsem1)
        pltpu.async_copy(table_hbm.at[srcb.at[0]], rows0, gsem0)

        def quad_body(i, carry):
            for u in range(4):
                j = 4 * i + u
                b = u % 2
                b1 = 1 - b
                rb, db, wbuf = rows[b], dstb[u], wb[b]
                pltpu.make_async_copy(table_hbm.at[srcb.at[j]], rb,
                                      gsem[b]).wait()
                pltpu.make_async_copy(dst_hbm.at[c0 + j], db, dsem[u]).wait()
                pltpu.make_async_copy(w_hbm.at[c0 + j], wbuf, wsem[b]).wait()

                scale_half(rb, wbuf, 0)
                scat_start(rb, db, 0, ssem[b])

                def drain_prev():
                    scat_wait(rows[b1], dstb[(u - 1) % 4], 0, ssem[b1])
                    scat_wait(rows[b1], dstb[(u - 1) % 4], 1, ssem[b1])

                if u == 0:
                    @pl.when(i > 0)
                    def _():
                        drain_prev()
                else:
                    drain_prev()

                if u < 3:
                    pltpu.async_copy(table_hbm.at[srcb.at[j + 1]],
                                     rows[b1], gsem[b1])
                else:
                    @pl.when(i < nch // 4 - 1)
                    def _():
                        pltpu.async_copy(table_hbm.at[srcb.at[j + 1]],
                                         rows[b1], gsem[b1])

                def dst_pf():
                    pltpu.async_copy(dst_hbm.at[c0 + j + 3],
                                     dstb[(u + 3) % 4], dsem[(u + 3) % 4])

                if u == 0:
                    dst_pf()
                else:
                    @pl.when(i < nch // 4 - 1)
                    def _():
                        dst_pf()

                scale_half(rb, wbuf, 1)
                scat_start(rb, db, 1, ssem[b])

                def w_pf():
                    pltpu.async_copy(w_hbm.at[c0 + j + 2], wbuf, wsem[b])

                if u < 2:
                    w_pf()
                else:
                    @pl.when(i < nch // 4 - 1)
                    def _():
                        w_pf()

            return carry

        lax.fori_loop(0, nch // 4, quad_body, 0)
        scat_wait(rows[1], dstb3, 0, ssem[1])
        scat_wait(rows[1], dstb3, 1, ssem[1])
        plsc.subcore_barrier()

        @pl.when(s < ns - 1)
        def _():
            pltpu.sync_copy(acc.at[pl.ds(r0, rpt)],
                            out_hbm.at[c, pl.ds(r0, rpt)])

        @pl.when(s == ns - 1)
        def _():
            pltpu.sync_copy(acc.at[pl.ds(r0, last_rows)],
                            out_hbm.at[c, pl.ds(r0, last_rows)])

    return spmm


def _tc_stage1(p, wcat, bcat):
    _, n, d = p.shape
    blk = 400
    assert n % blk == 0

    def body(p_ref, w_ref, b_ref, h1_ref, h2_ref):
        h = p_ref[0] + p_ref[1]
        y = jnp.dot(h, w_ref[...], preferred_element_type=jnp.float32)
        y = jnp.maximum(y + b_ref[...], 0.0)
        h1_ref[...] = y[:, :d]
        h2_ref[...] = y[:, d:]

    return pl.pallas_call(
        body,
        grid=(n // blk,),
        in_specs=[
            pl.BlockSpec((2, blk, d), lambda i: (0, i, 0)),
            pl.BlockSpec((d, 2 * d), lambda i: (0, 0)),
            pl.BlockSpec((2 * d,), lambda i: (0,)),
        ],
        out_specs=[
            pl.BlockSpec((blk, d), lambda i: (i, 0)),
            pl.BlockSpec((blk, d), lambda i: (i, 0)),
        ],
        out_shape=[
            jax.ShapeDtypeStruct((n, d), jnp.float32),
            jax.ShapeDtypeStruct((n, d), jnp.float32),
        ],
    )(p, wcat, bcat)


def _tc_stage2(pb, pc, w1, b1, w2, b2):
    _, n, d = pb.shape
    blk = 400
    assert n % blk == 0

    def body(pb_ref, pc_ref, w1_ref, b1_ref, w2_ref, b2_ref, x1_ref, x2_ref):
        s1 = pb_ref[0] + pb_ref[1]
        s2 = pc_ref[0] + pc_ref[1]
        x1_ref[...] = jnp.dot(s1, w1_ref[...],
                              preferred_element_type=jnp.float32) + b1_ref[...]
        x2_ref[...] = jnp.dot(s2, w2_ref[...],
                              preferred_element_type=jnp.float32) + b2_ref[...]

    return pl.pallas_call(
        body,
        grid=(n // blk,),
        in_specs=[
            pl.BlockSpec((2, blk, d), lambda i: (0, i, 0)),
            pl.BlockSpec((2, blk, d), lambda i: (0, i, 0)),
            pl.BlockSpec((d, d), lambda i: (0, 0)),
            pl.BlockSpec((d,), lambda i: (0,)),
            pl.BlockSpec((d, d), lambda i: (0, 0)),
            pl.BlockSpec((d,), lambda i: (0,)),
        ],
        out_specs=[
            pl.BlockSpec((blk, d), lambda i: (i, 0)),
            pl.BlockSpec((blk, d), lambda i: (i, 0)),
        ],
        out_shape=[
            jax.ShapeDtypeStruct((n, d), jnp.float32),
            jax.ShapeDtypeStruct((n, d), jnp.float32),
        ],
    )(pb, pc, w1, b1, w2, b2)


def kernel(x, edge_index, edge_weight, W1_0, b1_0, W1_1, b1_1,
           W2_0, b2_0, W2_1, b2_1):
    n, d = x.shape
    e = edge_weight.shape[0]

    info = plsc.get_sparse_core_info()
    nw = info.num_cores * info.num_subcores
    quantum = nw * _G * 4
    e_pad = ((e + quantum - 1) // quantum) * quantum
    pad = e_pad - e
    pad_idx = jnp.arange(pad, dtype=jnp.int32) % n
    dst = jnp.concatenate([edge_index[0], pad_idx]).reshape(
        e_pad // _G, 2, _G // 2)
    src = jnp.concatenate([edge_index[1], pad_idx]).reshape(e_pad // _G, _G)
    w = jnp.pad(edge_weight, (0, pad)).reshape(e_pad // _G, _G)
    zeros = jnp.zeros((n, d), jnp.float32)

    spmm = _spmm_partials_kernel(n, d, e_pad)
    pa = spmm(x, src, dst, w, zeros)

    wcat = jnp.concatenate([W1_0, W2_0], axis=1)
    bcat = jnp.concatenate([b1_0, b2_0], axis=0)
    h1, h2 = _tc_stage1(pa, wcat, bcat)

    pb = spmm(h1, src, dst, w, zeros)
    pc = spmm(h2, src, dst, w, zeros)

    x1, x2 = _tc_stage2(pb, pc, W1_1, b1_1, W2_1, b2_1)
    return (x1, x2)
